# baseline (speedup 1.0000x reference)
import jax
import jax.numpy as jnp
from jax import lax
from jax.experimental import pallas as pl
from jax.experimental.pallas import tpu as pltpu
from jax.experimental.pallas import tpu_sc as plsc

N = 10000
E = 320000
D_IN = 128
D_EDGE = 16
D_HID = 256
N_CLASSES = 128
EPSV = 1e-5

NC, NS = 2, 16
B = 128
NPAD = 10240
EPAD = 327680
RPS = NPAD // NS
NB_A = EPAD // (NC * NS * B)
NB_B = EPAD // (NS * B)
G = 16
NG_A = NB_A // G
NG_B = NB_B // G
RB = 256
GRID = NPAD // RB

_MESH = plsc.VectorSubcoreMesh(core_axis_name="c", subcore_axis_name="s")


def _scatter_edge_split(table, srci, dsti, z128):
  def body(table, srci, dsti, z128, out_s,
           acc_s, src_v, dst_v, rows0, rows1, gsem0, gsem1):
    c = lax.axis_index("c")
    s = lax.axis_index("s")
    row0 = s * RPS
    pltpu.sync_copy(z128.at[pl.ds(row0, RPS)], acc_s.at[pl.ds(row0, RPS)])
    plsc.subcore_barrier()

    def group(gi, carry):
      pltpu.sync_copy(srci.at[c, s, pl.ds(gi * G, G)], src_v)
      pltpu.sync_copy(dsti.at[c, s, pl.ds(gi * G, G)], dst_v)
      pltpu.async_copy(table.at[c].at[src_v.at[0]], rows0, gsem0)
      for g in range(G):
        buf, gsem = (rows0, gsem0) if g % 2 == 0 else (rows1, gsem1)
        nbuf, ngsem = (rows1, gsem1) if g % 2 == 0 else (rows0, gsem0)
        if g + 1 < G:
          pltpu.async_copy(table.at[c].at[src_v.at[g + 1]], nbuf, ngsem)
        pltpu.make_async_copy(table.at[c].at[src_v.at[g]], buf, gsem).wait()
        pltpu.sync_copy(buf, acc_s.at[dst_v.at[g]], add=True)
      return carry

    lax.fori_loop(0, NG_A, group, 0)
    plsc.subcore_barrier()
    pltpu.sync_copy(acc_s.at[pl.ds(row0, RPS)], out_s.at[c, pl.ds(row0, RPS)])

  fn = pl.kernel(
      body,
      out_type=jax.ShapeDtypeStruct((NC, NPAD, 128), jnp.float32),
      mesh=_MESH,
      scratch_types=[
          pltpu.VMEM_SHARED((NPAD, 128), jnp.float32),
          pltpu.VMEM((G, B), jnp.int32),
          pltpu.VMEM((G, B), jnp.int32),
          pltpu.VMEM((B, 128), jnp.float32),
          pltpu.VMEM((B, 128), jnp.float32),
          pltpu.SemaphoreType.DMA,
          pltpu.SemaphoreType.DMA,
      ],
  )
  return fn(jnp.broadcast_to(table, (NC,) + table.shape), srci, dsti, z128)


def _scatter_ea(ea, dsti, z128):
  def body(ea, dsti, z128, out_s, acc_s, dst_v, rows0, rows1, gsem0, gsem1):
    c = lax.axis_index("c")
    s = lax.axis_index("s")
    row0 = s * RPS
    pltpu.sync_copy(z128.at[pl.ds(row0, RPS)], acc_s.at[pl.ds(row0, RPS)])
    plsc.subcore_barrier()

    def group(gi, carry):
      pltpu.sync_copy(dsti.at[c, s, pl.ds(gi * G, G)], dst_v)
      pltpu.async_copy(ea.at[c, s, gi * G], rows0, gsem0)
      for g in range(G):
        buf, gsem = (rows0, gsem0) if g % 2 == 0 else (rows1, gsem1)
        nbuf, ngsem = (rows1, gsem1) if g % 2 == 0 else (rows0, gsem0)
        if g + 1 < G:
          pltpu.async_copy(ea.at[c, s, gi * G + g + 1], nbuf, ngsem)
        pltpu.make_async_copy(ea.at[c, s, gi * G + g], buf, gsem).wait()
        pltpu.sync_copy(buf, acc_s.at[dst_v.at[g]], add=True)
      return carry

    lax.fori_loop(0, NG_A, group, 0)
    plsc.subcore_barrier()
    pltpu.sync_copy(acc_s.at[pl.ds(row0, RPS)], out_s.at[c, pl.ds(row0, RPS)])

  fn = pl.kernel(
      body,
      out_type=jax.ShapeDtypeStruct((NC, NPAD, 128), jnp.float32),
      mesh=_MESH,
      scratch_types=[
          pltpu.VMEM_SHARED((NPAD, 128), jnp.float32),
          pltpu.VMEM((G, B), jnp.int32),
          pltpu.VMEM((B, 128), jnp.float32),
          pltpu.VMEM((B, 128), jnp.float32),
          pltpu.SemaphoreType.DMA,
          pltpu.SemaphoreType.DMA,
      ],
  )
  return fn(ea, dsti, z128)


def _scatter_col_split(table, srci, dsti, z128):
  def body(table, srci, dsti, z128, out_s,
           acc_s, src_v, dst_v, rows0, rows1, gsem0, gsem1):
    c = lax.axis_index("c")
    s = lax.axis_index("s")
    row0 = s * RPS
    pltpu.sync_copy(z128.at[pl.ds(row0, RPS)], acc_s.at[pl.ds(row0, RPS)])
    plsc.subcore_barrier()

    def group(gi, carry):
      pltpu.sync_copy(srci.at[s, pl.ds(gi * G, G)], src_v)
      pltpu.sync_copy(dsti.at[s, pl.ds(gi * G, G)], dst_v)
      pltpu.async_copy(table.at[c].at[src_v.at[0]], rows0, gsem0)
      for g in range(G):
        buf, gsem = (rows0, gsem0) if g % 2 == 0 else (rows1, gsem1)
        nbuf, ngsem = (rows1, gsem1) if g % 2 == 0 else (rows0, gsem0)
        if g + 1 < G:
          pltpu.async_copy(table.at[c].at[src_v.at[g + 1]], nbuf, ngsem)
        pltpu.make_async_copy(table.at[c].at[src_v.at[g]], buf, gsem).wait()
        pltpu.sync_copy(buf, acc_s.at[dst_v.at[g]], add=True)
      return carry

    lax.fori_loop(0, NG_B, group, 0)
    plsc.subcore_barrier()
    pltpu.sync_copy(acc_s.at[pl.ds(row0, RPS)], out_s.at[c, pl.ds(row0, RPS)])

  fn = pl.kernel(
      body,
      out_type=jax.ShapeDtypeStruct((NC, NPAD, 128), jnp.float32),
      mesh=_MESH,
      scratch_types=[
          pltpu.VMEM_SHARED((NPAD, 128), jnp.float32),
          pltpu.VMEM((G, B), jnp.int32),
          pltpu.VMEM((G, B), jnp.int32),
          pltpu.VMEM((B, 128), jnp.float32),
          pltpu.VMEM((B, 128), jnp.float32),
          pltpu.SemaphoreType.DMA,
          pltpu.SemaphoreType.DMA,
      ],
  )
  return fn(table, srci, dsti, z128)


def _ln_relu(y, g, be):
  mu = jnp.mean(y, axis=-1, keepdims=True)
  var = jnp.mean((y - mu) ** 2, axis=-1, keepdims=True)
  return jnp.maximum((y - mu) * lax.rsqrt(var + EPSV) * g + be, 0.0)


def _post0_body(sp_ref, x_ref, eap_ref, wh_ref, we_ref, bp_ref, g_ref, be_ref,
                out_ref):
  s = sp_ref[0] + sp_ref[1] + x_ref[...]
  a = eap_ref[0] + eap_ref[1]
  y = jnp.maximum(
      jnp.dot(s, wh_ref[...], preferred_element_type=jnp.float32)
      + jnp.dot(a, we_ref[...], preferred_element_type=jnp.float32)
      + bp_ref[...], 0.0)
  z = _ln_relu(y, g_ref[...], be_ref[...])
  out_ref[0] = z[:, :128]
  out_ref[1] = z[:, 128:]


def _post1_body(sp_ref, x_ref, eap_ref, wh_ref, we_ref, bp_ref, g_ref, be_ref,
                wh2_ref, out_ref):
  s = jnp.concatenate([sp_ref[0] + x_ref[0], sp_ref[1] + x_ref[1]], axis=1)
  a = eap_ref[0] + eap_ref[1]
  y = jnp.maximum(
      jnp.dot(s, wh_ref[...], preferred_element_type=jnp.float32)
      + jnp.dot(a, we_ref[...], preferred_element_type=jnp.float32)
      + bp_ref[...], 0.0)
  x2 = _ln_relu(y, g_ref[...], be_ref[...])
  out_ref[...] = jnp.dot(x2, wh2_ref[...], preferred_element_type=jnp.float32)


def _final_body(sp_ref, z_ref, eap_ref, we_ref, bp_ref, g_ref, be_ref, out_ref):
  s = sp_ref[0] + sp_ref[1] + z_ref[...]
  a = eap_ref[0] + eap_ref[1]
  y = jnp.maximum(
      s + jnp.dot(a, we_ref[...], preferred_element_type=jnp.float32)
      + bp_ref[...], 0.0)
  out_ref[...] = _ln_relu(y, g_ref[...], be_ref[...])


def kernel(h, edge_index, edge_attr, W0, b0, W1, b1, W2, b2,
           g0, be0, g1, be1, g2, be2):
  f32 = jnp.float32
  src = edge_index[0].astype(jnp.int32)
  dst = edge_index[1].astype(jnp.int32)
  pad_e = EPAD - E
  srcp = jnp.concatenate([src, jnp.full((pad_e,), N, jnp.int32)])
  dstp = jnp.concatenate([dst, jnp.full((pad_e,), N, jnp.int32)])
  src_a = srcp.reshape(NC, NS, NB_A, B)
  dst_a = dstp.reshape(NC, NS, NB_A, B)
  src_b = srcp.reshape(NS, NB_B, B)
  dst_b = dstp.reshape(NS, NB_B, B)
  ea128 = jnp.pad(edge_attr, ((0, pad_e), (0, 128 - D_EDGE)))
  ea_a = ea128.reshape(NC, NS, NB_A, B, 128)

  h_pad = jnp.zeros((NPAD, D_IN), f32).at[:N].set(h)
  z128 = jnp.zeros((NPAD, 128), f32)

  we0 = jnp.pad(W0[:D_EDGE], ((0, 128 - D_EDGE), (0, 0)))
  we1 = jnp.pad(W1[:D_EDGE], ((0, 128 - D_EDGE), (0, 0)))
  we2 = jnp.pad(W2[:D_EDGE], ((0, 128 - D_EDGE), (0, 0)))
  wh0, wh1, wh2 = W0[D_EDGE:], W1[D_EDGE:], W2[D_EDGE:]
  b0p = (b0 + W0[D_EDGE - 1]).reshape(1, -1)
  b1p = (b1 + W1[D_EDGE - 1]).reshape(1, -1)
  b2p = (b2 + W2[D_EDGE - 1]).reshape(1, -1)
  g0r, be0r = g0.reshape(1, -1), be0.reshape(1, -1)
  g1r, be1r = g1.reshape(1, -1), be1.reshape(1, -1)
  g2r, be2r = g2.reshape(1, -1), be2.reshape(1, -1)

  full2 = lambda r, c: pl.BlockSpec((r, c), lambda i: (0, 0))

  ea_p = _scatter_ea(ea_a, dst_a, z128)

  s0p = _scatter_edge_split(h_pad, src_a, dst_a, z128)
  x1cols = pl.pallas_call(
      _post0_body,
      grid=(GRID,),
      in_specs=[
          pl.BlockSpec((2, RB, 128), lambda i: (0, i, 0)),
          pl.BlockSpec((RB, 128), lambda i: (i, 0)),
          pl.BlockSpec((2, RB, 128), lambda i: (0, i, 0)),
          full2(D_IN, D_HID), full2(128, D_HID),
          full2(1, D_HID), full2(1, D_HID), full2(1, D_HID),
      ],
      out_specs=pl.BlockSpec((2, RB, 128), lambda i: (0, i, 0)),
      out_shape=jax.ShapeDtypeStruct((2, NPAD, 128), f32),
  )(s0p, h_pad, ea_p, wh0, we0, b0p, g0r, be0r)

  s1cols = _scatter_col_split(x1cols, src_b, dst_b, z128)
  z2 = pl.pallas_call(
      _post1_body,
      grid=(GRID,),
      in_specs=[
          pl.BlockSpec((2, RB, 128), lambda i: (0, i, 0)),
          pl.BlockSpec((2, RB, 128), lambda i: (0, i, 0)),
          pl.BlockSpec((2, RB, 128), lambda i: (0, i, 0)),
          full2(D_HID, D_HID), full2(128, D_HID),
          full2(1, D_HID), full2(1, D_HID), full2(1, D_HID),
          full2(D_HID, N_CLASSES),
      ],
      out_specs=pl.BlockSpec((RB, N_CLASSES), lambda i: (i, 0)),
      out_shape=jax.ShapeDtypeStruct((NPAD, N_CLASSES), f32),
  )(s1cols, x1cols, ea_p, wh1, we1, b1p, g1r, be1r, wh2)

  s2p = _scatter_edge_split(z2, src_a, dst_a, z128)
  out = pl.pallas_call(
      _final_body,
      grid=(GRID,),
      in_specs=[
          pl.BlockSpec((2, RB, N_CLASSES), lambda i: (0, i, 0)),
          pl.BlockSpec((RB, N_CLASSES), lambda i: (i, 0)),
          pl.BlockSpec((2, RB, 128), lambda i: (0, i, 0)),
          full2(128, N_CLASSES),
          full2(1, N_CLASSES), full2(1, N_CLASSES), full2(1, N_CLASSES),
      ],
      out_specs=pl.BlockSpec((RB, N_CLASSES), lambda i: (i, 0)),
      out_shape=jax.ShapeDtypeStruct((NPAD, N_CLASSES), f32),
  )(s2p, z2, ea_p, we2, b2p, g2r, be2r)

  return out[:N]


# speedup vs baseline: 1.0501x; 1.0501x over previous
import jax
import jax.numpy as jnp
from jax import lax
from jax.experimental import pallas as pl
from jax.experimental.pallas import tpu as pltpu
from jax.experimental.pallas import tpu_sc as plsc

N = 10000
E = 320000
D_IN = 128
D_EDGE = 16
D_HID = 256
N_CLASSES = 128
EPSV = 1e-5

NC, NS = 2, 16
B = 64
NBUF = 4
NPAD = 10240
EPAD = 327680
RPS = NPAD // NS
NB_A = EPAD // (NC * NS * B)
NB_B = EPAD // (NS * B)
G = 16
NG_A = NB_A // G
NG_B = NB_B // G
RB = 256
GRID = NPAD // RB

_MESH = plsc.VectorSubcoreMesh(core_axis_name="c", subcore_axis_name="s")


def _scatter_edge_split(table, srci, dsti, z128):
  def body(table, srci, dsti, z128, out_s, acc_s, src_v, dst_v, *bufs):
    rows = bufs[:NBUF]
    gsem = bufs[NBUF:]
    c = lax.axis_index("c")
    s = lax.axis_index("s")
    row0 = s * RPS
    pltpu.sync_copy(z128.at[pl.ds(row0, RPS)], acc_s.at[pl.ds(row0, RPS)])
    plsc.subcore_barrier()

    def group(gi, carry):
      pltpu.sync_copy(srci.at[c, s, pl.ds(gi * G, G)], src_v)
      pltpu.sync_copy(dsti.at[c, s, pl.ds(gi * G, G)], dst_v)
      for k in range(NBUF - 1):
        pltpu.async_copy(table.at[src_v.at[k]], rows[k], gsem[k])
      for g in range(G):
        p = g % NBUF
        if g + NBUF - 1 < G:
          q = (g + NBUF - 1) % NBUF
          pltpu.async_copy(table.at[src_v.at[g + NBUF - 1]], rows[q], gsem[q])
        pltpu.make_async_copy(table.at[src_v.at[g]], rows[p], gsem[p]).wait()
        pltpu.sync_copy(rows[p], acc_s.at[dst_v.at[g]], add=True)
      return carry

    lax.fori_loop(0, NG_A, group, 0)
    plsc.subcore_barrier()
    pltpu.sync_copy(acc_s.at[pl.ds(row0, RPS)], out_s.at[c, pl.ds(row0, RPS)])

  fn = pl.kernel(
      body,
      out_type=jax.ShapeDtypeStruct((NC, NPAD, 128), jnp.float32),
      mesh=_MESH,
      scratch_types=[
          pltpu.VMEM_SHARED((NPAD, 128), jnp.float32),
          pltpu.VMEM((G, B), jnp.int32),
          pltpu.VMEM((G, B), jnp.int32),
      ] + [pltpu.VMEM((B, 128), jnp.float32)] * NBUF
        + [pltpu.SemaphoreType.DMA] * NBUF,
  )
  return fn(table, srci, dsti, z128)


def _scatter_ea(ea, dsti, z128):
  def body(ea, dsti, z128, out_s, acc_s, dst_v, *bufs):
    rows = bufs[:NBUF]
    gsem = bufs[NBUF:]
    c = lax.axis_index("c")
    s = lax.axis_index("s")
    row0 = s * RPS
    pltpu.sync_copy(z128.at[pl.ds(row0, RPS)], acc_s.at[pl.ds(row0, RPS)])
    plsc.subcore_barrier()

    def group(gi, carry):
      pltpu.sync_copy(dsti.at[c, s, pl.ds(gi * G, G)], dst_v)
      for k in range(NBUF - 1):
        pltpu.async_copy(ea.at[c, s, gi * G + k], rows[k], gsem[k])
      for g in range(G):
        p = g % NBUF
        if g + NBUF - 1 < G:
          q = (g + NBUF - 1) % NBUF
          pltpu.async_copy(ea.at[c, s, gi * G + g + NBUF - 1], rows[q], gsem[q])
        pltpu.make_async_copy(ea.at[c, s, gi * G + g], rows[p], gsem[p]).wait()
        pltpu.sync_copy(rows[p], acc_s.at[dst_v.at[g]], add=True)
      return carry

    lax.fori_loop(0, NG_A, group, 0)
    plsc.subcore_barrier()
    pltpu.sync_copy(acc_s.at[pl.ds(row0, RPS)], out_s.at[c, pl.ds(row0, RPS)])

  fn = pl.kernel(
      body,
      out_type=jax.ShapeDtypeStruct((NC, NPAD, 128), jnp.float32),
      mesh=_MESH,
      scratch_types=[
          pltpu.VMEM_SHARED((NPAD, 128), jnp.float32),
          pltpu.VMEM((G, B), jnp.int32),
      ] + [pltpu.VMEM((B, 128), jnp.float32)] * NBUF
        + [pltpu.SemaphoreType.DMA] * NBUF,
  )
  return fn(ea, dsti, z128)


def _scatter_col_split(table, srci, dsti, z128):
  def body(table, srci, dsti, z128, out_s, acc_s, src_v, dst_v, *bufs):
    rows = bufs[:NBUF]
    gsem = bufs[NBUF:]
    c = lax.axis_index("c")
    s = lax.axis_index("s")
    row0 = s * RPS
    pltpu.sync_copy(z128.at[pl.ds(row0, RPS)], acc_s.at[pl.ds(row0, RPS)])
    plsc.subcore_barrier()

    def group(gi, carry):
      pltpu.sync_copy(srci.at[s, pl.ds(gi * G, G)], src_v)
      pltpu.sync_copy(dsti.at[s, pl.ds(gi * G, G)], dst_v)
      for k in range(NBUF - 1):
        pltpu.async_copy(table.at[c].at[src_v.at[k]], rows[k], gsem[k])
      for g in range(G):
        p = g % NBUF
        if g + NBUF - 1 < G:
          q = (g + NBUF - 1) % NBUF
          pltpu.async_copy(table.at[c].at[src_v.at[g + NBUF - 1]],
                           rows[q], gsem[q])
        pltpu.make_async_copy(table.at[c].at[src_v.at[g]],
                              rows[p], gsem[p]).wait()
        pltpu.sync_copy(rows[p], acc_s.at[dst_v.at[g]], add=True)
      return carry

    lax.fori_loop(0, NG_B, group, 0)
    plsc.subcore_barrier()
    pltpu.sync_copy(acc_s.at[pl.ds(row0, RPS)], out_s.at[c, pl.ds(row0, RPS)])

  fn = pl.kernel(
      body,
      out_type=jax.ShapeDtypeStruct((NC, NPAD, 128), jnp.float32),
      mesh=_MESH,
      scratch_types=[
          pltpu.VMEM_SHARED((NPAD, 128), jnp.float32),
          pltpu.VMEM((G, B), jnp.int32),
          pltpu.VMEM((G, B), jnp.int32),
      ] + [pltpu.VMEM((B, 128), jnp.float32)] * NBUF
        + [pltpu.SemaphoreType.DMA] * NBUF,
  )
  return fn(table, srci, dsti, z128)


def _ln_relu(y, g, be):
  mu = jnp.mean(y, axis=-1, keepdims=True)
  var = jnp.mean((y - mu) ** 2, axis=-1, keepdims=True)
  return jnp.maximum((y - mu) * lax.rsqrt(var + EPSV) * g + be, 0.0)


def _post0_body(sp_ref, x_ref, eap_ref, wh_ref, we_ref, bp_ref, g_ref, be_ref,
                out_ref):
  s = sp_ref[0] + sp_ref[1] + x_ref[...]
  a = eap_ref[0] + eap_ref[1]
  y = jnp.maximum(
      jnp.dot(s, wh_ref[...], preferred_element_type=jnp.float32)
      + jnp.dot(a, we_ref[...], preferred_element_type=jnp.float32)
      + bp_ref[...], 0.0)
  z = _ln_relu(y, g_ref[...], be_ref[...])
  out_ref[0] = z[:, :128]
  out_ref[1] = z[:, 128:]


def _post1_body(sp_ref, x_ref, eap_ref, wh_ref, we_ref, bp_ref, g_ref, be_ref,
                wh2_ref, out_ref):
  s = jnp.concatenate([sp_ref[0] + x_ref[0], sp_ref[1] + x_ref[1]], axis=1)
  a = eap_ref[0] + eap_ref[1]
  y = jnp.maximum(
      jnp.dot(s, wh_ref[...], preferred_element_type=jnp.float32)
      + jnp.dot(a, we_ref[...], preferred_element_type=jnp.float32)
      + bp_ref[...], 0.0)
  x2 = _ln_relu(y, g_ref[...], be_ref[...])
  out_ref[...] = jnp.dot(x2, wh2_ref[...], preferred_element_type=jnp.float32)


def _final_body(sp_ref, z_ref, eap_ref, we_ref, bp_ref, g_ref, be_ref, out_ref):
  s = sp_ref[0] + sp_ref[1] + z_ref[...]
  a = eap_ref[0] + eap_ref[1]
  y = jnp.maximum(
      s + jnp.dot(a, we_ref[...], preferred_element_type=jnp.float32)
      + bp_ref[...], 0.0)
  out_ref[...] = _ln_relu(y, g_ref[...], be_ref[...])


def kernel(h, edge_index, edge_attr, W0, b0, W1, b1, W2, b2,
           g0, be0, g1, be1, g2, be2):
  f32 = jnp.float32
  src = edge_index[0].astype(jnp.int32)
  dst = edge_index[1].astype(jnp.int32)
  pad_e = EPAD - E
  srcp = jnp.concatenate([src, jnp.full((pad_e,), N, jnp.int32)])
  dstp = jnp.concatenate([dst, jnp.full((pad_e,), N, jnp.int32)])
  src_a = srcp.reshape(NC, NS, NB_A, B)
  dst_a = dstp.reshape(NC, NS, NB_A, B)
  src_b = srcp.reshape(NS, NB_B, B)
  dst_b = dstp.reshape(NS, NB_B, B)
  ea128 = jnp.pad(edge_attr, ((0, pad_e), (0, 128 - D_EDGE)))
  ea_a = ea128.reshape(NC, NS, NB_A, B, 128)

  h_pad = jnp.zeros((NPAD, D_IN), f32).at[:N].set(h)
  z128 = jnp.zeros((NPAD, 128), f32)

  we0 = jnp.pad(W0[:D_EDGE], ((0, 128 - D_EDGE), (0, 0)))
  we1 = jnp.pad(W1[:D_EDGE], ((0, 128 - D_EDGE), (0, 0)))
  we2 = jnp.pad(W2[:D_EDGE], ((0, 128 - D_EDGE), (0, 0)))
  wh0, wh1, wh2 = W0[D_EDGE:], W1[D_EDGE:], W2[D_EDGE:]
  b0p = (b0 + W0[D_EDGE - 1]).reshape(1, -1)
  b1p = (b1 + W1[D_EDGE - 1]).reshape(1, -1)
  b2p = (b2 + W2[D_EDGE - 1]).reshape(1, -1)
  g0r, be0r = g0.reshape(1, -1), be0.reshape(1, -1)
  g1r, be1r = g1.reshape(1, -1), be1.reshape(1, -1)
  g2r, be2r = g2.reshape(1, -1), be2.reshape(1, -1)

  full2 = lambda r, c: pl.BlockSpec((r, c), lambda i: (0, 0))

  ea_p = _scatter_ea(ea_a, dst_a, z128)

  s0p = _scatter_edge_split(h_pad, src_a, dst_a, z128)
  x1cols = pl.pallas_call(
      _post0_body,
      grid=(GRID,),
      in_specs=[
          pl.BlockSpec((2, RB, 128), lambda i: (0, i, 0)),
          pl.BlockSpec((RB, 128), lambda i: (i, 0)),
          pl.BlockSpec((2, RB, 128), lambda i: (0, i, 0)),
          full2(D_IN, D_HID), full2(128, D_HID),
          full2(1, D_HID), full2(1, D_HID), full2(1, D_HID),
      ],
      out_specs=pl.BlockSpec((2, RB, 128), lambda i: (0, i, 0)),
      out_shape=jax.ShapeDtypeStruct((2, NPAD, 128), f32),
  )(s0p, h_pad, ea_p, wh0, we0, b0p, g0r, be0r)

  s1cols = _scatter_col_split(x1cols, src_b, dst_b, z128)
  z2 = pl.pallas_call(
      _post1_body,
      grid=(GRID,),
      in_specs=[
          pl.BlockSpec((2, RB, 128), lambda i: (0, i, 0)),
          pl.BlockSpec((2, RB, 128), lambda i: (0, i, 0)),
          pl.BlockSpec((2, RB, 128), lambda i: (0, i, 0)),
          full2(D_HID, D_HID), full2(128, D_HID),
          full2(1, D_HID), full2(1, D_HID), full2(1, D_HID),
          full2(D_HID, N_CLASSES),
      ],
      out_specs=pl.BlockSpec((RB, N_CLASSES), lambda i: (i, 0)),
      out_shape=jax.ShapeDtypeStruct((NPAD, N_CLASSES), f32),
  )(s1cols, x1cols, ea_p, wh1, we1, b1p, g1r, be1r, wh2)

  s2p = _scatter_edge_split(z2, src_a, dst_a, z128)
  out = pl.pallas_call(
      _final_body,
      grid=(GRID,),
      in_specs=[
          pl.BlockSpec((2, RB, N_CLASSES), lambda i: (0, i, 0)),
          pl.BlockSpec((RB, N_CLASSES), lambda i: (i, 0)),
          pl.BlockSpec((2, RB, 128), lambda i: (0, i, 0)),
          full2(128, N_CLASSES),
          full2(1, N_CLASSES), full2(1, N_CLASSES), full2(1, N_CLASSES),
      ],
      out_specs=pl.BlockSpec((RB, N_CLASSES), lambda i: (i, 0)),
      out_shape=jax.ShapeDtypeStruct((NPAD, N_CLASSES), f32),
  )(s2p, z2, ea_p, we2, b2p, g2r, be2r)

  return out[:N]


# speedup vs baseline: 1.9016x; 1.8109x over previous
import jax
import jax.numpy as jnp
from jax import lax
from jax.experimental import pallas as pl
from jax.experimental.pallas import tpu as pltpu
from jax.experimental.pallas import tpu_sc as plsc

N = 10000
E = 320000
D_IN = 128
D_EDGE = 16
D_HID = 256
N_CLASSES = 128
EPSV = 1e-5

NC, NS = 2, 16
B = 128
NBUF = 2
W_BLK = 64
NPAD = 10240
EPAD = 327680
RPS = NPAD // NS
NB_A = EPAD // (NC * NS * B)
NB_B = EPAD // (NS * B)
G = 16
NG_A = NB_A // G
NG_B = NB_B // G
RB = 256
GRID = NPAD // RB

_MESH = plsc.VectorSubcoreMesh(core_axis_name="c", subcore_axis_name="s")


def _scatter_blocks(table, srci, dsti):
  nblk = table.shape[0]
  per_sc = nblk // NC

  def body(table, srci, dsti, out_s, spm_tab, acc_s, src_v, dst_v, *bufs):
    rows = bufs[:NBUF]
    gsem = bufs[NBUF:]
    c = lax.axis_index("c")
    s = lax.axis_index("s")
    row0 = s * RPS
    for k in range(per_sc):
      blk = c * per_sc + k
      pltpu.sync_copy(table.at[blk, pl.ds(row0, RPS)],
                      spm_tab.at[pl.ds(row0, RPS)])
      pltpu.sync_copy(table.at[blk, pl.ds(row0, RPS)],
                      acc_s.at[pl.ds(row0, RPS)])
      plsc.subcore_barrier()

      def group(gi, carry):
        pltpu.sync_copy(srci.at[s, pl.ds(gi * G, G)], src_v)
        pltpu.sync_copy(dsti.at[s, pl.ds(gi * G, G)], dst_v)
        for j in range(NBUF - 1):
          pltpu.async_copy(spm_tab.at[src_v.at[j]], rows[j], gsem[j])
        for g in range(G):
          p = g % NBUF
          if g + NBUF - 1 < G:
            q = (g + NBUF - 1) % NBUF
            pltpu.async_copy(spm_tab.at[src_v.at[g + NBUF - 1]],
                             rows[q], gsem[q])
          pltpu.make_async_copy(spm_tab.at[src_v.at[g]],
                                rows[p], gsem[p]).wait()
          pltpu.sync_copy(rows[p], acc_s.at[dst_v.at[g]], add=True)
        return carry

      lax.fori_loop(0, NG_B, group, 0)
      plsc.subcore_barrier()
      pltpu.sync_copy(acc_s.at[pl.ds(row0, RPS)],
                      out_s.at[blk, pl.ds(row0, RPS)])
      plsc.subcore_barrier()

  fn = pl.kernel(
      body,
      out_type=jax.ShapeDtypeStruct((nblk, NPAD, W_BLK), jnp.float32),
      mesh=_MESH,
      scratch_types=[
          pltpu.VMEM_SHARED((NPAD, W_BLK), jnp.float32),
          pltpu.VMEM_SHARED((NPAD, W_BLK), jnp.float32),
          pltpu.VMEM((G, B), jnp.int32),
          pltpu.VMEM((G, B), jnp.int32),
      ] + [pltpu.VMEM((B, W_BLK), jnp.float32)] * NBUF
        + [pltpu.SemaphoreType.DMA] * NBUF,
  )
  return fn(table, srci, dsti)


def _scatter_ea(ea, dsti, z128):
  def body(ea, dsti, z128, out_s, acc_s, dst_v, *bufs):
    rows = bufs[:NBUF]
    gsem = bufs[NBUF:]
    c = lax.axis_index("c")
    s = lax.axis_index("s")
    row0 = s * RPS
    pltpu.sync_copy(z128.at[pl.ds(row0, RPS)], acc_s.at[pl.ds(row0, RPS)])
    plsc.subcore_barrier()

    def group(gi, carry):
      pltpu.sync_copy(dsti.at[c, s, pl.ds(gi * G, G)], dst_v)
      for k in range(NBUF - 1):
        pltpu.async_copy(ea.at[c, s, gi * G + k], rows[k], gsem[k])
      for g in range(G):
        p = g % NBUF
        if g + NBUF - 1 < G:
          q = (g + NBUF - 1) % NBUF
          pltpu.async_copy(ea.at[c, s, gi * G + g + NBUF - 1], rows[q], gsem[q])
        pltpu.make_async_copy(ea.at[c, s, gi * G + g], rows[p], gsem[p]).wait()
        pltpu.sync_copy(rows[p], acc_s.at[dst_v.at[g]], add=True)
      return carry

    lax.fori_loop(0, NG_A, group, 0)
    plsc.subcore_barrier()
    pltpu.sync_copy(acc_s.at[pl.ds(row0, RPS)], out_s.at[c, pl.ds(row0, RPS)])

  fn = pl.kernel(
      body,
      out_type=jax.ShapeDtypeStruct((NC, NPAD, 128), jnp.float32),
      mesh=_MESH,
      scratch_types=[
          pltpu.VMEM_SHARED((NPAD, 128), jnp.float32),
          pltpu.VMEM((G, B), jnp.int32),
      ] + [pltpu.VMEM((B, 128), jnp.float32)] * NBUF
        + [pltpu.SemaphoreType.DMA] * NBUF,
  )
  return fn(ea, dsti, z128)


def _ln_relu(y, g, be):
  mu = jnp.mean(y, axis=-1, keepdims=True)
  var = jnp.mean((y - mu) ** 2, axis=-1, keepdims=True)
  return jnp.maximum((y - mu) * lax.rsqrt(var + EPSV) * g + be, 0.0)


def _post0_body(sp_ref, eap_ref, wh_ref, we_ref, bp_ref, g_ref, be_ref,
                out_ref):
  s = jnp.concatenate([sp_ref[0], sp_ref[1]], axis=1)
  a = eap_ref[0] + eap_ref[1]
  y = jnp.maximum(
      jnp.dot(s, wh_ref[...], preferred_element_type=jnp.float32)
      + jnp.dot(a, we_ref[...], preferred_element_type=jnp.float32)
      + bp_ref[...], 0.0)
  z = _ln_relu(y, g_ref[...], be_ref[...])
  for k in range(4):
    out_ref[k] = z[:, k * W_BLK:(k + 1) * W_BLK]


def _post1_body(sp_ref, eap_ref, wh_ref, we_ref, bp_ref, g_ref, be_ref,
                wh2_ref, out_ref):
  s = jnp.concatenate([sp_ref[0], sp_ref[1], sp_ref[2], sp_ref[3]], axis=1)
  a = eap_ref[0] + eap_ref[1]
  y = jnp.maximum(
      jnp.dot(s, wh_ref[...], preferred_element_type=jnp.float32)
      + jnp.dot(a, we_ref[...], preferred_element_type=jnp.float32)
      + bp_ref[...], 0.0)
  x2 = _ln_relu(y, g_ref[...], be_ref[...])
  z2 = jnp.dot(x2, wh2_ref[...], preferred_element_type=jnp.float32)
  for k in range(2):
    out_ref[k] = z2[:, k * W_BLK:(k + 1) * W_BLK]


def _final_body(sp_ref, eap_ref, we_ref, bp_ref, g_ref, be_ref, out_ref):
  s = jnp.concatenate([sp_ref[0], sp_ref[1]], axis=1)
  a = eap_ref[0] + eap_ref[1]
  y = jnp.maximum(
      s + jnp.dot(a, we_ref[...], preferred_element_type=jnp.float32)
      + bp_ref[...], 0.0)
  out_ref[...] = _ln_relu(y, g_ref[...], be_ref[...])


def kernel(h, edge_index, edge_attr, W0, b0, W1, b1, W2, b2,
           g0, be0, g1, be1, g2, be2):
  f32 = jnp.float32
  src = edge_index[0].astype(jnp.int32)
  dst = edge_index[1].astype(jnp.int32)
  pad_e = EPAD - E
  srcp = jnp.concatenate([src, jnp.full((pad_e,), N, jnp.int32)])
  dstp = jnp.concatenate([dst, jnp.full((pad_e,), N, jnp.int32)])
  src_a = srcp.reshape(NC, NS, NB_A, B)
  dst_a = dstp.reshape(NC, NS, NB_A, B)
  src_b = srcp.reshape(NS, NB_B, B)
  dst_b = dstp.reshape(NS, NB_B, B)
  ea128 = jnp.pad(edge_attr, ((0, pad_e), (0, 128 - D_EDGE)))
  ea_a = ea128.reshape(NC, NS, NB_A, B, 128)

  h_pad = jnp.zeros((NPAD, D_IN), f32).at[:N].set(h)
  h_blk = h_pad.reshape(NPAD, 2, W_BLK).transpose(1, 0, 2)
  z128 = jnp.zeros((NPAD, 128), f32)

  we0 = jnp.pad(W0[:D_EDGE], ((0, 128 - D_EDGE), (0, 0)))
  we1 = jnp.pad(W1[:D_EDGE], ((0, 128 - D_EDGE), (0, 0)))
  we2 = jnp.pad(W2[:D_EDGE], ((0, 128 - D_EDGE), (0, 0)))
  wh0, wh1, wh2 = W0[D_EDGE:], W1[D_EDGE:], W2[D_EDGE:]
  b0p = (b0 + W0[D_EDGE - 1]).reshape(1, -1)
  b1p = (b1 + W1[D_EDGE - 1]).reshape(1, -1)
  b2p = (b2 + W2[D_EDGE - 1]).reshape(1, -1)
  g0r, be0r = g0.reshape(1, -1), be0.reshape(1, -1)
  g1r, be1r = g1.reshape(1, -1), be1.reshape(1, -1)
  g2r, be2r = g2.reshape(1, -1), be2.reshape(1, -1)

  full2 = lambda r, c: pl.BlockSpec((r, c), lambda i: (0, 0))

  ea_p = _scatter_ea(ea_a, dst_a, z128)

  s0 = _scatter_blocks(h_blk, src_b, dst_b)
  x1blk = pl.pallas_call(
      _post0_body,
      grid=(GRID,),
      in_specs=[
          pl.BlockSpec((2, RB, W_BLK), lambda i: (0, i, 0)),
          pl.BlockSpec((2, RB, 128), lambda i: (0, i, 0)),
          full2(D_IN, D_HID), full2(128, D_HID),
          full2(1, D_HID), full2(1, D_HID), full2(1, D_HID),
      ],
      out_specs=pl.BlockSpec((4, RB, W_BLK), lambda i: (0, i, 0)),
      out_shape=jax.ShapeDtypeStruct((4, NPAD, W_BLK), f32),
  )(s0, ea_p, wh0, we0, b0p, g0r, be0r)

  s1 = _scatter_blocks(x1blk, src_b, dst_b)
  z2blk = pl.pallas_call(
      _post1_body,
      grid=(GRID,),
      in_specs=[
          pl.BlockSpec((4, RB, W_BLK), lambda i: (0, i, 0)),
          pl.BlockSpec((2, RB, 128), lambda i: (0, i, 0)),
          full2(D_HID, D_HID), full2(128, D_HID),
          full2(1, D_HID), full2(1, D_HID), full2(1, D_HID),
          full2(D_HID, N_CLASSES),
      ],
      out_specs=pl.BlockSpec((2, RB, W_BLK), lambda i: (0, i, 0)),
      out_shape=jax.ShapeDtypeStruct((2, NPAD, W_BLK), f32),
  )(s1, ea_p, wh1, we1, b1p, g1r, be1r, wh2)

  s2 = _scatter_blocks(z2blk, src_b, dst_b)
  out = pl.pallas_call(
      _final_body,
      grid=(GRID,),
      in_specs=[
          pl.BlockSpec((2, RB, W_BLK), lambda i: (0, i, 0)),
          pl.BlockSpec((2, RB, 128), lambda i: (0, i, 0)),
          full2(128, N_CLASSES),
          full2(1, N_CLASSES), full2(1, N_CLASSES), full2(1, N_CLASSES),
      ],
      out_specs=pl.BlockSpec((RB, N_CLASSES), lambda i: (i, 0)),
      out_shape=jax.ShapeDtypeStruct((NPAD, N_CLASSES), f32),
  )(s2, ea_p, we2, b2p, g2r, be2r)

  return out[:N]
